# baseline (speedup 1.0000x reference)
import functools

import jax
import jax.numpy as jnp
from jax import lax
from jax.experimental import pallas as pl
from jax.experimental.pallas import tpu as pltpu
from jax.experimental.pallas import tpu_sc as plsc

NC = 2
NS = 16
NW = NC * NS
LANES = 16
K = 16
NXB = 2
NPB = 4


def _pe_add_body(seq, batch, dim, seq_per_w,
                 x_hbm, idx_hbm, pe_hbm, out_hbm,
                 idx_v, xbuf, pebuf,
                 sx0, sx1, sp0, sp1, sp2, sp3, so0, so1, so2, so3):
    sxs = [sx0, sx1]
    sps = [sp0, sp1, sp2, sp3]
    sos = [so0, so1, so2, so3]
    nchunks = seq_per_w // K

    wid = lax.axis_index("s") * NC + lax.axis_index("c")
    wpb = NW // batch
    b = wid // wpb
    s0 = (wid % wpb) * seq_per_w

    pltpu.sync_copy(idx_hbm.at[b, pl.ds(s0, seq_per_w)], idx_v)

    def start_x(c, u):
        pltpu.make_async_copy(
            x_hbm.at[pl.ds(s0 + c * K, K), b], xbuf.at[u % NXB], sxs[u % NXB]
        ).start()

    def wait_x(u):
        pltpu.make_async_copy(
            x_hbm.at[pl.ds(s0, K), b], xbuf.at[u % NXB], sxs[u % NXB]
        ).wait()

    def start_pe(c, u):
        pltpu.make_async_copy(
            pe_hbm.at[idx_v.at[pl.ds(c * K, K)]], pebuf.at[u % NPB],
            sps[u % NPB]
        ).start()

    def wait_pe(u):
        pltpu.make_async_copy(
            pe_hbm.at[idx_v.at[pl.ds(0, K)]], pebuf.at[u % NPB], sps[u % NPB]
        ).wait()

    def start_out(c, u):
        pltpu.make_async_copy(
            pebuf.at[u % NPB], out_hbm.at[pl.ds(s0 + c * K, K), b],
            sos[u % NPB]
        ).start()

    def wait_out(u):
        pltpu.make_async_copy(
            pebuf.at[u % NPB], out_hbm.at[pl.ds(s0, K), b], sos[u % NPB]
        ).wait()

    def add_chunk(u):
        xq, pq = u % NXB, u % NPB

        def add_row(j, _):
            for i in range(dim // LANES):
                v = xbuf[xq, j, pl.ds(i * LANES, LANES)]
                plsc.addupdate(pebuf.at[pq, j, pl.ds(i * LANES, LANES)], v)
            return 0

        lax.fori_loop(0, K, add_row, 0)

    def chunk(c, u, head, tail):
        if not head:
            wait_out(u + 2)
        if not tail:
            start_pe(c + 2, u + 2)
        wait_x(u)
        wait_pe(u)
        add_chunk(u)
        if not tail:
            start_x(c + 2, u)
        start_out(c, u)

    start_x(0, 0)
    start_x(1, 1)
    start_pe(0, 0)
    start_pe(1, 1)

    for u in range(4):
        chunk(u, u, head=(u < 2), tail=False)

    def group(g, _):
        c0 = g * 4
        for u in range(4):
            chunk(c0 + u, u, head=False, tail=False)
        return 0

    lax.fori_loop(1, nchunks // 4 - 1, group, 0)

    cl = nchunks - 4
    for u in range(4):
        c = cl + u
        wait_out(u + 2)
        if u < 2:
            start_pe(c + 2, u + 2)
        wait_x(u)
        wait_pe(u)
        add_chunk(u)
        if u < 2:
            start_x(c + 2, u)
        start_out(c, u)

    wait_out(2)
    wait_out(3)


def kernel(x, indices, pe):
    seq, batch, dim = x.shape
    seq_per_w = seq // (NW // batch)

    idx = indices.astype(jnp.int32)

    mesh = plsc.VectorSubcoreMesh(core_axis_name="c", subcore_axis_name="s")
    body = functools.partial(_pe_add_body, seq, batch, dim, seq_per_w)
    f = pl.kernel(
        body,
        mesh=mesh,
        out_type=jax.ShapeDtypeStruct((seq, batch, dim), jnp.float32),
        scratch_types=[
            pltpu.VMEM((seq_per_w,), jnp.int32),
            pltpu.VMEM((NXB, K, dim), jnp.float32),
            pltpu.VMEM((NPB, K, dim), jnp.float32),
        ] + [pltpu.SemaphoreType.DMA] * (NXB + NPB + NPB),
    )
    return f(x, idx, pe)


# speedup vs baseline: 1.0726x; 1.0726x over previous
import functools

import jax
import jax.numpy as jnp
from jax import lax
from jax.experimental import pallas as pl
from jax.experimental.pallas import tpu as pltpu
from jax.experimental.pallas import tpu_sc as plsc

NC = 2
NS = 16
NW = NC * NS
LANES = 16
K = 8
NXB = 4
NPB = 8


def _pe_add_body(seq, batch, dim, seq_per_w,
                 x_hbm, idx_hbm, pe_hbm, out_hbm,
                 idx_v, xbuf, pebuf, *sems):
    sxs = sems[:NXB]
    sps = sems[NXB:NXB + NPB]
    sos = sems[NXB + NPB:]
    nchunks = seq_per_w // K

    wid = lax.axis_index("s") * NC + lax.axis_index("c")
    wpb = NW // batch
    b = wid // wpb
    s0 = (wid % wpb) * seq_per_w

    pltpu.sync_copy(idx_hbm.at[b, pl.ds(s0, seq_per_w)], idx_v)

    def start_x(c, u):
        pltpu.make_async_copy(
            x_hbm.at[pl.ds(s0 + c * K, K), b], xbuf.at[u % NXB], sxs[u % NXB]
        ).start()

    def wait_x(u):
        pltpu.make_async_copy(
            x_hbm.at[pl.ds(s0, K), b], xbuf.at[u % NXB], sxs[u % NXB]
        ).wait()

    def start_pe(c, u):
        pltpu.make_async_copy(
            pe_hbm.at[idx_v.at[pl.ds(c * K, K)]], pebuf.at[u % NPB],
            sps[u % NPB]
        ).start()

    def wait_pe(u):
        pltpu.make_async_copy(
            pe_hbm.at[idx_v.at[pl.ds(0, K)]], pebuf.at[u % NPB], sps[u % NPB]
        ).wait()

    def start_out(c, u):
        pltpu.make_async_copy(
            pebuf.at[u % NPB], out_hbm.at[pl.ds(s0 + c * K, K), b],
            sos[u % NPB]
        ).start()

    def wait_out(u):
        pltpu.make_async_copy(
            pebuf.at[u % NPB], out_hbm.at[pl.ds(s0, K), b], sos[u % NPB]
        ).wait()

    def add_chunk(u):
        xq, pq = u % NXB, u % NPB

        def add_row(j, _):
            for i in range(dim // LANES):
                v = xbuf[xq, j, pl.ds(i * LANES, LANES)]
                plsc.addupdate(pebuf.at[pq, j, pl.ds(i * LANES, LANES)], v)
            return 0

        lax.fori_loop(0, K, add_row, 0)

    G = 8
    PF = 3
    DL = 5

    def chunk(c, u, head, tail):
        if not head or u >= DL:
            wait_out(u - DL)
        if not tail or u < G - PF:
            start_pe(c + PF, u + PF)
            start_x(c + PF, u + PF)
        wait_x(u)
        wait_pe(u)
        add_chunk(u)
        start_out(c, u)

    for c in range(PF):
        start_x(c, c)
        start_pe(c, c)

    for u in range(G):
        chunk(u, u, head=True, tail=False)

    def group(g, _):
        c0 = g * G
        for u in range(G):
            chunk(c0 + u, u, head=False, tail=False)
        return 0

    lax.fori_loop(1, nchunks // G - 1, group, 0)

    cl = nchunks - G
    for u in range(G):
        chunk(cl + u, u, head=False, tail=True)

    for u in range(G - DL, G):
        wait_out(u)


def kernel(x, indices, pe):
    seq, batch, dim = x.shape
    seq_per_w = seq // (NW // batch)

    idx = indices.astype(jnp.int32)

    mesh = plsc.VectorSubcoreMesh(core_axis_name="c", subcore_axis_name="s")
    body = functools.partial(_pe_add_body, seq, batch, dim, seq_per_w)
    f = pl.kernel(
        body,
        mesh=mesh,
        out_type=jax.ShapeDtypeStruct((seq, batch, dim), jnp.float32),
        scratch_types=[
            pltpu.VMEM((seq_per_w,), jnp.int32),
            pltpu.VMEM((NXB, K, dim), jnp.float32),
            pltpu.VMEM((NPB, K, dim), jnp.float32),
        ] + [pltpu.SemaphoreType.DMA] * (NXB + NPB + NPB),
    )
    return f(x, idx, pe)
